# baseline (speedup 1.0000x reference)
import functools

import jax
import jax.numpy as jnp
from jax import lax
from jax.experimental import pallas as pl
from jax.experimental.pallas import tpu as pltpu
from jax.experimental.pallas import tpu_sc as plsc

N = 10000
E = 320000
D = 128
K = 128
NC = 2
NS = 16
W = NC * NS
NP = 10240
ROWS_PER_TILE = NP // NS
EPW = 10240
CH = EPW // K
E_PAD = W * EPW - E
NSL = 4


def _sc_segment_sum(x, eidx, zeros):
    mesh = plsc.VectorSubcoreMesh(core_axis_name="c", subcore_axis_name="s")

    @functools.partial(
        pl.kernel,
        mesh=mesh,
        out_type=jax.ShapeDtypeStruct((NC, NP, D), jnp.float32),
        scratch_types=[
            pltpu.VMEM((NSL, 2, K), jnp.int32),
            pltpu.VMEM((K, D), jnp.float32),
            pltpu.VMEM((K, D), jnp.float32),
            pltpu.VMEM_SHARED((NP, D), jnp.float32),
            pltpu.SemaphoreType.DMA,
            pltpu.SemaphoreType.DMA,
            [pltpu.SemaphoreType.DMA] * NSL,
        ],
        compiler_params=pltpu.CompilerParams(use_tc_tiling_on_sc=False),
    )
    def body(x_hbm, eidx_hbm, zeros_hbm, out_hbm, idx, rows0, rows1, agg_sh,
             sem0, sem1, isem):
        c = lax.axis_index("c")
        s = lax.axis_index("s")
        wid = s * NC + c
        rbuf = (rows0, rows1)
        gsem = (sem0, sem1)
        row0 = s * ROWS_PER_TILE
        for m in range(NSL):
            pltpu.async_copy(eidx_hbm.at[wid, m], idx.at[m], isem[m])
        pltpu.sync_copy(zeros_hbm.at[pl.ds(row0, ROWS_PER_TILE)],
                        agg_sh.at[pl.ds(row0, ROWS_PER_TILE)])
        plsc.subcore_barrier()
        for b in range(2):
            pltpu.make_async_copy(eidx_hbm.at[wid, 0], idx.at[b],
                                  isem[b]).wait()
            pltpu.async_copy(x_hbm.at[idx.at[b, 0]], rbuf[b], gsem[b])

        def quad(g, carry):
            c0 = NSL * g
            for j in range(NSL):
                b = j % 2
                m = j % NSL
                pltpu.make_async_copy(x_hbm.at[pl.ds(0, K)], rbuf[b],
                                      gsem[b]).wait()
                pltpu.sync_copy(rbuf[b], agg_sh.at[idx.at[m, 1]], add=True)
                pltpu.async_copy(eidx_hbm.at[wid, c0 + j + NSL], idx.at[m],
                                 isem[m])
                m2 = (j + 2) % NSL
                pltpu.make_async_copy(eidx_hbm.at[wid, 0], idx.at[m2],
                                      isem[m2]).wait()
                pltpu.async_copy(x_hbm.at[idx.at[m2, 0]], rbuf[b], gsem[b])
            return carry

        lax.fori_loop(0, CH // NSL - 1, quad, 0)
        for j in range(NSL):
            b = j % 2
            pltpu.make_async_copy(x_hbm.at[pl.ds(0, K)], rbuf[b],
                                  gsem[b]).wait()
            pltpu.sync_copy(rbuf[b], agg_sh.at[idx.at[j, 1]], add=True)
            if j < 2:
                m2 = j + 2
                pltpu.make_async_copy(eidx_hbm.at[wid, 0], idx.at[m2],
                                      isem[m2]).wait()
                pltpu.async_copy(x_hbm.at[idx.at[m2, 0]], rbuf[b], gsem[b])
        plsc.subcore_barrier()
        pltpu.sync_copy(agg_sh.at[pl.ds(row0, ROWS_PER_TILE)],
                        out_hbm.at[c, pl.ds(row0, ROWS_PER_TILE)])

    return body(x, eidx, zeros)


def _dgru_block(part_ref, x_ref, Ww_ref, Wb_ref, Uw_ref, Ub_ref,
                lng_ref, lnb_ref, ln2g_ref, ln2b_ref, out_ref):
    agg = part_ref[0] + part_ref[1]
    h = x_ref[...]
    inp = jnp.concatenate([agg, h], axis=1)
    mu = jnp.mean(inp, axis=1, keepdims=True)
    cent = inp - mu
    var = jnp.mean(cent * cent, axis=1, keepdims=True)
    inp = cent * lax.rsqrt(var + 1e-5) * lng_ref[...] + lnb_ref[...]
    gates = jnp.dot(inp, Ww_ref[...], preferred_element_type=jnp.float32)
    gates = gates + Wb_ref[...]
    g0 = gates[:, 0 * D:1 * D]
    g1 = gates[:, 1 * D:2 * D]
    g2 = gates[:, 2 * D:3 * D]
    g3 = gates[:, 3 * D:4 * D]
    g4 = gates[:, 4 * D:5 * D]
    rx = jax.nn.sigmoid(g0)
    rh = jax.nn.sigmoid(g1)
    m = jnp.maximum(jnp.maximum(g2, g3), g4)
    e2 = jnp.exp(g2 - m)
    e3 = jnp.exp(g3 - m)
    e4 = jnp.exp(g4 - m)
    zs = e2 + e3 + e4
    inp2 = jnp.concatenate([agg * rx, h * rh], axis=1)
    mu2 = jnp.mean(inp2, axis=1, keepdims=True)
    cent2 = inp2 - mu2
    var2 = jnp.mean(cent2 * cent2, axis=1, keepdims=True)
    inp2 = cent2 * lax.rsqrt(var2 + 1e-5) * ln2g_ref[...] + ln2b_ref[...]
    u = jnp.tanh(jnp.dot(inp2, Uw_ref[...], preferred_element_type=jnp.float32)
                 + Ub_ref[...])
    out_ref[...] = (agg * e2 + h * e3 + u * e4) / zs


def _dgru(part, x, gateW_w, gateW_b, gateU_w, gateU_b, ln_g, ln_b, ln2_g, ln2_b):
    R = 1000
    grid = (N // R,)
    return pl.pallas_call(
        _dgru_block,
        grid=grid,
        in_specs=[
            pl.BlockSpec((NC, R, D), lambda i: (0, i, 0)),
            pl.BlockSpec((R, D), lambda i: (i, 0)),
            pl.BlockSpec((2 * D, 5 * D), lambda i: (0, 0)),
            pl.BlockSpec((1, 5 * D), lambda i: (0, 0)),
            pl.BlockSpec((2 * D, D), lambda i: (0, 0)),
            pl.BlockSpec((1, D), lambda i: (0, 0)),
            pl.BlockSpec((1, 2 * D), lambda i: (0, 0)),
            pl.BlockSpec((1, 2 * D), lambda i: (0, 0)),
            pl.BlockSpec((1, 2 * D), lambda i: (0, 0)),
            pl.BlockSpec((1, 2 * D), lambda i: (0, 0)),
        ],
        out_specs=pl.BlockSpec((R, D), lambda i: (i, 0)),
        out_shape=jax.ShapeDtypeStruct((N, D), jnp.float32),
    )(part, x, gateW_w, gateW_b.reshape(1, -1), gateU_w,
      gateU_b.reshape(1, -1), ln_g.reshape(1, -1), ln_b.reshape(1, -1),
      ln2_g.reshape(1, -1), ln2_b.reshape(1, -1))


def kernel(x, edge_index, gateW_w, gateW_b, gateU_w, gateU_b,
           ln_g, ln_b, ln2_g, ln2_b):
    src_p = jnp.concatenate(
        [edge_index[0], jnp.zeros((E_PAD,), jnp.int32)]).reshape(W, CH, K)
    dst_p = jnp.concatenate(
        [edge_index[1], jnp.full((E_PAD,), N, jnp.int32)]).reshape(W, CH, K)
    eidx = jnp.stack([src_p, dst_p], axis=2)
    zeros = jnp.zeros((NP, D), jnp.float32)
    part = _sc_segment_sum(x, eidx, zeros)
    return _dgru(part, x, gateW_w, gateW_b, gateU_w, gateU_b,
                 ln_g, ln_b, ln2_g, ln2_b)


# speedup vs baseline: 3.1593x; 3.1593x over previous
import functools

import jax
import jax.numpy as jnp
from jax import lax
from jax.experimental import pallas as pl
from jax.experimental.pallas import tpu as pltpu
from jax.experimental.pallas import tpu_sc as plsc

N = 10000
E = 320000
D = 128
K = 80
NC = 2
NS = 16
W = NC * NS
NP = 10240
ROWS_PER_TILE = NP // NS
CH = E // (W * K)


def _sc_segment_sum(x, src_r, dst_r, zeros):
    mesh = plsc.VectorSubcoreMesh(core_axis_name="c", subcore_axis_name="s")

    @functools.partial(
        pl.kernel,
        mesh=mesh,
        out_type=jax.ShapeDtypeStruct((NC, NP, D), jnp.float32),
        scratch_types=[
            pltpu.VMEM((CH, K), jnp.int32),
            pltpu.VMEM((CH, K), jnp.int32),
            pltpu.VMEM((K, D), jnp.float32),
            pltpu.VMEM((K, D), jnp.float32),
            pltpu.VMEM_SHARED((NP, D), jnp.float32),
            pltpu.SemaphoreType.DMA,
            pltpu.SemaphoreType.DMA,
        ],
        compiler_params=pltpu.CompilerParams(use_tc_tiling_on_sc=False),
    )
    def body(x_hbm, src_hbm, dst_hbm, zeros_hbm, out_hbm, src_v, dst_v,
             rows0, rows1, agg_sh, sem0, sem1):
        c = lax.axis_index("c")
        s = lax.axis_index("s")
        wid = s * NC + c
        pltpu.sync_copy(src_hbm.at[wid], src_v)
        pltpu.sync_copy(dst_hbm.at[wid], dst_v)
        row0 = s * ROWS_PER_TILE
        pltpu.async_copy(x_hbm.at[src_v.at[0]], rows0, sem0)
        pltpu.async_copy(x_hbm.at[src_v.at[1]], rows1, sem1)
        pltpu.sync_copy(zeros_hbm, agg_sh.at[pl.ds(row0, ROWS_PER_TILE)])
        plsc.subcore_barrier()

        def pair(g, carry):
            c0 = 2 * g
            pltpu.make_async_copy(x_hbm.at[pl.ds(0, K)], rows0, sem0).wait()
            pltpu.sync_copy(rows0, agg_sh.at[dst_v.at[c0]], add=True)
            pltpu.async_copy(x_hbm.at[src_v.at[c0 + 2]], rows0, sem0)
            pltpu.make_async_copy(x_hbm.at[pl.ds(0, K)], rows1, sem1).wait()
            pltpu.sync_copy(rows1, agg_sh.at[dst_v.at[c0 + 1]], add=True)
            pltpu.async_copy(x_hbm.at[src_v.at[c0 + 3]], rows1, sem1)
            return carry

        lax.fori_loop(0, (CH - 3) // 2, pair, 0)
        pltpu.make_async_copy(x_hbm.at[pl.ds(0, K)], rows0, sem0).wait()
        pltpu.sync_copy(rows0, agg_sh.at[dst_v.at[CH - 3]], add=True)
        pltpu.async_copy(x_hbm.at[src_v.at[CH - 1]], rows0, sem0)
        pltpu.make_async_copy(x_hbm.at[pl.ds(0, K)], rows1, sem1).wait()
        pltpu.sync_copy(rows1, agg_sh.at[dst_v.at[CH - 2]], add=True)
        pltpu.make_async_copy(x_hbm.at[pl.ds(0, K)], rows0, sem0).wait()
        pltpu.sync_copy(rows0, agg_sh.at[dst_v.at[CH - 1]], add=True)
        plsc.subcore_barrier()
        pltpu.sync_copy(agg_sh.at[pl.ds(row0, ROWS_PER_TILE)],
                        out_hbm.at[c, pl.ds(row0, ROWS_PER_TILE)])

    return body(x, src_r, dst_r, zeros)


def _dgru_block(part_ref, x_ref, Ww_ref, Wb_ref, Uw_ref, Ub_ref,
                lng_ref, lnb_ref, ln2g_ref, ln2b_ref, out_ref):
    agg = part_ref[0] + part_ref[1]
    h = x_ref[...]
    inp = jnp.concatenate([agg, h], axis=1)
    mu = jnp.mean(inp, axis=1, keepdims=True)
    cent = inp - mu
    var = jnp.mean(cent * cent, axis=1, keepdims=True)
    inp = cent * lax.rsqrt(var + 1e-5) * lng_ref[...] + lnb_ref[...]
    gates = jnp.dot(inp, Ww_ref[...], preferred_element_type=jnp.float32)
    gates = gates + Wb_ref[...]
    g0 = gates[:, 0 * D:1 * D]
    g1 = gates[:, 1 * D:2 * D]
    g2 = gates[:, 2 * D:3 * D]
    g3 = gates[:, 3 * D:4 * D]
    g4 = gates[:, 4 * D:5 * D]
    rx = jax.nn.sigmoid(g0)
    rh = jax.nn.sigmoid(g1)
    m = jnp.maximum(jnp.maximum(g2, g3), g4)
    e2 = jnp.exp(g2 - m)
    e3 = jnp.exp(g3 - m)
    e4 = jnp.exp(g4 - m)
    zs = e2 + e3 + e4
    inp2 = jnp.concatenate([agg * rx, h * rh], axis=1)
    mu2 = jnp.mean(inp2, axis=1, keepdims=True)
    cent2 = inp2 - mu2
    var2 = jnp.mean(cent2 * cent2, axis=1, keepdims=True)
    inp2 = cent2 * lax.rsqrt(var2 + 1e-5) * ln2g_ref[...] + ln2b_ref[...]
    u = jnp.tanh(jnp.dot(inp2, Uw_ref[...], preferred_element_type=jnp.float32)
                 + Ub_ref[...])
    out_ref[...] = (agg * e2 + h * e3 + u * e4) / zs


def _dgru(part, x, gateW_w, gateW_b, gateU_w, gateU_b, ln_g, ln_b, ln2_g, ln2_b):
    R = 2000
    grid = (N // R,)
    return pl.pallas_call(
        _dgru_block,
        grid=grid,
        in_specs=[
            pl.BlockSpec((NC, R, D), lambda i: (0, i, 0)),
            pl.BlockSpec((R, D), lambda i: (i, 0)),
            pl.BlockSpec((2 * D, 5 * D), lambda i: (0, 0)),
            pl.BlockSpec((1, 5 * D), lambda i: (0, 0)),
            pl.BlockSpec((2 * D, D), lambda i: (0, 0)),
            pl.BlockSpec((1, D), lambda i: (0, 0)),
            pl.BlockSpec((1, 2 * D), lambda i: (0, 0)),
            pl.BlockSpec((1, 2 * D), lambda i: (0, 0)),
            pl.BlockSpec((1, 2 * D), lambda i: (0, 0)),
            pl.BlockSpec((1, 2 * D), lambda i: (0, 0)),
        ],
        out_specs=pl.BlockSpec((R, D), lambda i: (i, 0)),
        out_shape=jax.ShapeDtypeStruct((N, D), jnp.float32),
    )(part, x, gateW_w, gateW_b.reshape(1, -1), gateU_w,
      gateU_b.reshape(1, -1), ln_g.reshape(1, -1), ln_b.reshape(1, -1),
      ln2_g.reshape(1, -1), ln2_b.reshape(1, -1))


def kernel(x, edge_index, gateW_w, gateW_b, gateU_w, gateU_b,
           ln_g, ln_b, ln2_g, ln2_b):
    src_p = edge_index[0].reshape(W, CH, K)
    dst_p = edge_index[1].reshape(W, CH, K)
    zeros = jnp.zeros((ROWS_PER_TILE, D), jnp.float32)
    part = _sc_segment_sum(x, src_p, dst_p, zeros)
    return _dgru(part, x, gateW_w, gateW_b, gateU_w, gateU_b,
                 ln_g, ln_b, ln2_g, ln2_b)
